# baseline (speedup 1.0000x reference)
import functools

import jax
import jax.numpy as jnp
from jax import lax
from jax.experimental import pallas as pl
from jax.experimental.pallas import tpu as pltpu
from jax.experimental.pallas import tpu_sc as plsc

EMBED_DIM = 64
SEQ = 50
LANES = 128


@functools.cache
def _build(T: int, V: int):
    info = plsc.get_sparse_core_info()
    nw = info.num_cores * info.num_subcores
    assert T == nw * LANES
    n_planes = SEQ

    mesh = plsc.VectorSubcoreMesh(core_axis_name="c", subcore_axis_name="s")

    @functools.partial(
        pl.kernel,
        mesh=mesh,
        compiler_params=pltpu.CompilerParams(
            use_tc_tiling_on_sc=False, needs_layout_passes=False),
        out_type=jax.ShapeDtypeStruct(
            (SEQ, EMBED_DIM // 8, nw, 8, LANES), jnp.float32),
        scratch_types=[
            pltpu.VMEM((1, n_planes, LANES), jnp.int32),
            pltpu.VMEM((LANES, EMBED_DIM), jnp.float32),
            pltpu.VMEM((LANES, EMBED_DIM), jnp.float32),
            pltpu.VMEM((EMBED_DIM // 8, 8, LANES), jnp.float32),
            pltpu.VMEM((EMBED_DIM // 8, 8, LANES), jnp.float32),
        ]
        + [pltpu.SemaphoreType.DMA] * 4,
    )
    def gather_kernel(table_hbm, idx_hbm, out_hbm, idx_v, g0, g1, t0, t1,
                      gs0, gs1, ws0, ws1):
        gbufs = (g0, g1)
        tbufs = (t0, t1)
        gsems = (gs0, gs1)
        wsems = (ws0, ws1)
        wid = lax.axis_index("s") * info.num_cores + lax.axis_index("c")
        pltpu.sync_copy(idx_hbm.at[pl.ds(wid, 1)], idx_v)
        idx_rows = idx_v.at[0]

        def start_gather(j, b):
            pltpu.async_copy(
                table_hbm.at[idx_rows.at[j]], gbufs[b], gsems[b])

        def transpose_plane(b):
            gbuf, tbuf = gbufs[b], tbufs[b]

            @plsc.parallel_loop(0, EMBED_DIM, unroll=8)
            def _(d):
                dt = d // 8
                s = d % 8
                col = jnp.full((16,), 0, jnp.int32) + d
                for l0 in range(0, LANES, 16):
                    rows = l0 + lax.iota(jnp.int32, 16)
                    tbuf.at[dt, s][pl.ds(l0, 16)] = plsc.load_gather(
                        gbuf, [rows, col])

        start_gather(0, 0)
        start_gather(1, 1)

        def step(t, _):
            for b in range(2):
                j = t * 2 + b
                pltpu.make_async_copy(
                    table_hbm.at[idx_rows.at[j]], gbufs[b], gsems[b]).wait()
                @pl.when(j >= 2)
                def _():
                    pltpu.make_async_copy(
                        tbufs[b], out_hbm.at[j - 2, :, wid], wsems[b]).wait()

                transpose_plane(b)
                pltpu.async_copy(tbufs[b], out_hbm.at[j, :, wid], wsems[b])

                @pl.when(j + 2 < n_planes)
                def _():
                    start_gather(j + 2, b)
            return 0

        lax.fori_loop(0, n_planes // 2, step, 0)

        for j in (n_planes - 2, n_planes - 1):
            b = j % 2
            pltpu.make_async_copy(
                tbufs[b], out_hbm.at[j, :, wid], wsems[b]).wait()

    return gather_kernel


def kernel(token_ids, emb_matrix):
    T, S = token_ids.shape
    nw = 32
    idx3d = jnp.transpose(
        token_ids.astype(jnp.int32).T.reshape(S, nw, LANES), (1, 0, 2))
    x5 = _build(T, emb_matrix.shape[0])(emb_matrix, idx3d)
    return jnp.transpose(x5, (2, 4, 0, 1, 3)).reshape(T, S, EMBED_DIM)


# speedup vs baseline: 1.8808x; 1.8808x over previous
import functools

import jax
import jax.numpy as jnp
from jax import lax
from jax.experimental import pallas as pl
from jax.experimental.pallas import tpu as pltpu
from jax.experimental.pallas import tpu_sc as plsc

EMBED_DIM = 64
SEQ = 50
LANES = 128


@functools.cache
def _build(T: int, V: int):
    info = plsc.get_sparse_core_info()
    nw = info.num_cores * info.num_subcores
    assert T == nw * LANES
    n_planes = SEQ

    mesh = plsc.VectorSubcoreMesh(core_axis_name="c", subcore_axis_name="s")

    @functools.partial(
        pl.kernel,
        mesh=mesh,
        compiler_params=pltpu.CompilerParams(
            use_tc_tiling_on_sc=False, needs_layout_passes=False),
        out_type=jax.ShapeDtypeStruct(
            (SEQ, EMBED_DIM // 8, nw, 8, LANES), jnp.float32),
        scratch_types=[
            pltpu.VMEM((1, n_planes, LANES), jnp.int32),
            pltpu.VMEM((LANES, EMBED_DIM), jnp.float32),
            pltpu.VMEM((LANES, EMBED_DIM), jnp.float32),
            pltpu.VMEM((EMBED_DIM // 8, 8, LANES), jnp.float32),
            pltpu.VMEM((EMBED_DIM // 8, 8, LANES), jnp.float32),
        ]
        + [pltpu.SemaphoreType.DMA] * 4,
    )
    def gather_kernel(table_hbm, idx_hbm, out_hbm, idx_v, g0, g1, t0, t1,
                      gs0, gs1, ws0, ws1):
        gbufs = (g0, g1)
        tbufs = (t0, t1)
        gsems = (gs0, gs1)
        wsems = (ws0, ws1)
        wid = lax.axis_index("s") * info.num_cores + lax.axis_index("c")
        pltpu.sync_copy(idx_hbm.at[pl.ds(wid, 1)], idx_v)
        idx_rows = idx_v.at[0]

        def start_gather(j, b):
            pltpu.async_copy(
                table_hbm.at[idx_rows.at[j]], gbufs[b], gsems[b])

        def transpose_plane(b):
            gbuf, tbuf = gbufs[b], tbufs[b]
            iota = lax.iota(jnp.int32, 16)
            for k in range(16):
                rot = lax.bitwise_and(iota + k, 15)

                @plsc.parallel_loop(0, 32, unroll=4)
                def _(bl):
                    l0 = (bl % 8) * 16
                    c0 = (bl // 8) * 16
                    row = l0 + iota
                    d = c0 + rot
                    v = plsc.load_gather(gbuf, [row, d])
                    plsc.store_scatter(
                        tbuf,
                        [lax.shift_right_logical(d, 3),
                         lax.bitwise_and(d, 7), row],
                        v)

        start_gather(0, 0)
        start_gather(1, 1)

        def step(t, _):
            for b in range(2):
                j = t * 2 + b
                pltpu.make_async_copy(
                    table_hbm.at[idx_rows.at[j]], gbufs[b], gsems[b]).wait()
                @pl.when(j >= 2)
                def _():
                    pltpu.make_async_copy(
                        tbufs[b], out_hbm.at[j - 2, :, wid], wsems[b]).wait()

                transpose_plane(b)
                pltpu.async_copy(tbufs[b], out_hbm.at[j, :, wid], wsems[b])

                @pl.when(j + 2 < n_planes)
                def _():
                    start_gather(j + 2, b)
            return 0

        lax.fori_loop(0, n_planes // 2, step, 0)

        for j in (n_planes - 2, n_planes - 1):
            b = j % 2
            pltpu.make_async_copy(
                tbufs[b], out_hbm.at[j, :, wid], wsems[b]).wait()

    return gather_kernel


def kernel(token_ids, emb_matrix):
    T, S = token_ids.shape
    nw = 32
    idx3d = jnp.transpose(
        token_ids.astype(jnp.int32).T.reshape(S, nw, LANES), (1, 0, 2))
    x5 = _build(T, emb_matrix.shape[0])(emb_matrix, idx3d)
    return jnp.transpose(x5, (2, 4, 0, 1, 3)).reshape(T, S, EMBED_DIM)
